# baseline (speedup 1.0000x reference)
import jax
import jax.numpy as jnp
from jax.experimental import pallas as pl
from jax.experimental.pallas import tpu as pltpu

D = 512
H = 8
DH = 64
N = 16384
BLK = 1024
NB = N // BLK
OMPAD = 640
NEG = -1e30
SCALE = 0.125
F32 = jnp.float32


_SELU_SCALE = 1.0507009873554805
_SELU_ALPHA = 1.6732632423543772


def _selu(x):
    neg = _SELU_ALPHA * (jnp.exp(jnp.minimum(x, 0.0)) - 1.0)
    return _SELU_SCALE * jnp.where(x > 0, x, neg)


def _omic_kernel(xom_ref, w1_ref, b1_ref, w2_ref, b2_ref, out_ref):
    for i in range(6):
        h = xom_ref[i : i + 1, :] @ w1_ref[i]
        h = _selu(h + b1_ref[i : i + 1, :])
        h = _selu(h @ w2_ref[i] + b2_ref[i : i + 1, :])
        out_ref[i : i + 1, :] = h
    out_ref[6:8, :] = jnp.zeros((2, D), F32)


def _path_attn(h, q_w, q_b, kv, vv):
    q = h @ q_w + q_b
    outs = []
    for hh in range(H):
        sl = slice(hh * DH, (hh + 1) * DH)
        s = jax.lax.dot_general(
            q[:, sl], kv[:, sl], (((1,), (1,)), ((), ())),
            preferred_element_type=F32) * SCALE
        mask = jax.lax.broadcasted_iota(jnp.int32, s.shape, 1) >= 6
        s = jnp.where(mask, NEG, s)
        mx = jnp.max(s, axis=1, keepdims=True)
        e = jnp.exp(s - mx)
        a = e / jnp.sum(e, axis=1, keepdims=True)
        outs.append(a @ vv[:, sl])
    return jnp.concatenate(outs, axis=1)


def _stream_omic_attn(qo, kb, vb, m_scr, l_scr, acc_scr):
    for hh in range(H):
        sl = slice(hh * DH, (hh + 1) * DH)
        rs = slice(hh * 8, (hh + 1) * 8)
        s = jax.lax.dot_general(
            qo[:, sl], kb[:, sl], (((1,), (1,)), ((), ())),
            preferred_element_type=F32) * SCALE
        m_old = m_scr[rs, 0:1]
        m_new = jnp.maximum(m_old, jnp.max(s, axis=1, keepdims=True))
        alpha = jnp.exp(m_old - m_new)
        p = jnp.exp(s - m_new)
        l_scr[rs, 0:1] = l_scr[rs, 0:1] * alpha + jnp.sum(p, axis=1, keepdims=True)
        acc_scr[rs, :] = acc_scr[rs, :] * alpha + jnp.dot(
            p, vb[:, sl], preferred_element_type=F32)
        m_scr[rs, 0:1] = m_new


def _finish_omic_attn(m_scr, l_scr, acc_scr):
    outs = []
    for hh in range(H):
        rs = slice(hh * 8, (hh + 1) * 8)
        outs.append(acc_scr[rs, :] / l_scr[rs, 0:1])
    return jnp.concatenate(outs, axis=1)


def _passA_kernel(x_ref, wsiW_ref, wsib_ref, hom_ref,
                  wq0, bq0, wk0, bk0, wv0, bv0, wo0, bo0, wf0, bf0,
                  wq1, bq1, wk1, bk1, wv1, bv1, wo1, bo1, wf1, bf1,
                  h1_ref, hom1_ref,
                  k0_scr, v0_scr, q1_scr, m_scr, l_scr, acc_scr):
    i = pl.program_id(0)

    @pl.when(i == 0)
    def _init():
        hom = hom_ref[...]
        k0_scr[...] = hom @ wk0[...] + bk0[...]
        v0_scr[...] = hom @ wv0[...] + bv0[...]
        q1_scr[...] = hom @ wq1[...] + bq1[...]
        m_scr[...] = jnp.full(m_scr.shape, NEG, F32)
        l_scr[...] = jnp.zeros(l_scr.shape, F32)
        acc_scr[...] = jnp.zeros(acc_scr.shape, F32)

    h0 = jnp.maximum(x_ref[...] @ wsiW_ref[...] + wsib_ref[...], 0.0)
    o = _path_attn(h0, wq0[...], bq0[...], k0_scr[...], v0_scr[...])
    attn = o @ wo0[...] + bo0[...]
    h1 = jnp.maximum(attn @ wf0[...] + bf0[...], 0.0)
    h1_ref[...] = h1

    kb = h1 @ wk1[...] + bk1[...]
    vb = h1 @ wv1[...] + bv1[...]
    _stream_omic_attn(q1_scr[...], kb, vb, m_scr, l_scr, acc_scr)

    @pl.when(i == NB - 1)
    def _fin():
        o_om = _finish_omic_attn(m_scr, l_scr, acc_scr)
        attn_om = o_om @ wo1[...] + bo1[...]
        hom1_ref[...] = jnp.maximum(attn_om @ wf1[...] + bf1[...], 0.0)


def _passB_kernel(h1_ref, hom1_ref,
                  wq2, bq2, wk2, bk2, wv2, bv2, wo2, bo2, wf2, bf2,
                  wq3, bq3, wk3, bk3, wv3, bv3, wo3, bo3, wf3, bf3,
                  saV_ref, saw_ref, clsW_ref, clsb_ref,
                  out_ref,
                  k2_scr, v2_scr, q3_scr, m_scr, l_scr, acc_scr,
                  pm_scr, pl_scr, pacc_scr):
    i = pl.program_id(0)

    @pl.when(i == 0)
    def _init():
        hom = hom1_ref[...]
        k2_scr[...] = hom @ wk2[...] + bk2[...]
        v2_scr[...] = hom @ wv2[...] + bv2[...]
        q3_scr[...] = hom @ wq3[...] + bq3[...]
        m_scr[...] = jnp.full(m_scr.shape, NEG, F32)
        l_scr[...] = jnp.zeros(l_scr.shape, F32)
        acc_scr[...] = jnp.zeros(acc_scr.shape, F32)
        pm_scr[...] = jnp.full(pm_scr.shape, NEG, F32)
        pl_scr[...] = jnp.zeros(pl_scr.shape, F32)
        pacc_scr[...] = jnp.zeros(pacc_scr.shape, F32)

    h1 = h1_ref[...]
    o = _path_attn(h1, wq2[...], bq2[...], k2_scr[...], v2_scr[...])
    attn = o @ wo2[...] + bo2[...]
    h2 = jnp.maximum(attn @ wf2[...] + bf2[...], 0.0)

    t = jnp.tanh(h2 @ saV_ref[0])
    sT = jax.lax.dot_general(
        saw_ref[0:1, :], t, (((1,), (1,)), ((), ())),
        preferred_element_type=F32)
    m_old = pm_scr[0:1, 0:1]
    m_new = jnp.maximum(m_old, jnp.max(sT, axis=1, keepdims=True))
    alpha = jnp.exp(m_old - m_new)
    p = jnp.exp(sT - m_new)
    pl_scr[0:1, 0:1] = pl_scr[0:1, 0:1] * alpha + jnp.sum(p, axis=1, keepdims=True)
    pacc_scr[0:1, :] = pacc_scr[0:1, :] * alpha + jnp.dot(
        p, h2, preferred_element_type=F32)
    pm_scr[0:1, 0:1] = m_new

    kb = h2 @ wk3[...] + bk3[...]
    vb = h2 @ wv3[...] + bv3[...]
    _stream_omic_attn(q3_scr[...], kb, vb, m_scr, l_scr, acc_scr)

    @pl.when(i == NB - 1)
    def _fin():
        o_om = _finish_omic_attn(m_scr, l_scr, acc_scr)
        attn_om = o_om @ wo3[...] + bo3[...]
        hom_f = jnp.maximum(attn_om @ wf3[...] + bf3[...], 0.0)
        t2 = jnp.tanh(hom_f @ saV_ref[1])
        s2 = jax.lax.dot_general(
            saw_ref[1:2, :], t2, (((1,), (1,)), ((), ())),
            preferred_element_type=F32)
        mask = jax.lax.broadcasted_iota(jnp.int32, s2.shape, 1) >= 6
        s2 = jnp.where(mask, NEG, s2)
        mx = jnp.max(s2, axis=1, keepdims=True)
        e = jnp.exp(s2 - mx)
        a = e / jnp.sum(e, axis=1, keepdims=True)
        ho = jnp.dot(a, hom_f, preferred_element_type=F32)
        hp = pacc_scr[0:1, :] / pl_scr[0:1, 0:1]
        hcat = jnp.concatenate([hp, ho], axis=1)
        logits = hcat @ clsW_ref[...] + clsb_ref[...]
        out_ref[...] = jnp.broadcast_to(logits, out_ref.shape)


def _full(shape):
    return pl.BlockSpec(shape, lambda i: (0,) * len(shape))


def kernel(x_path, x_omic1, x_omic2, x_omic3, x_omic4, x_omic5, x_omic6,
           wsi_W, wsi_b,
           sig_W1_1, sig_W1_2, sig_W1_3, sig_W1_4, sig_W1_5, sig_W1_6,
           sig_b1, sig_W2, sig_b2,
           coa_Wq, coa_bq, coa_Wk, coa_bk, coa_Wv, coa_bv, coa_Wo, coa_bo,
           coa_Wf, coa_bf,
           sa_V, sa_w, cls_W, cls_b):
    x = x_path[0]

    omics = [x_omic1, x_omic2, x_omic3, x_omic4, x_omic5, x_omic6]
    w1s = [sig_W1_1, sig_W1_2, sig_W1_3, sig_W1_4, sig_W1_5, sig_W1_6]
    xom = jnp.stack([jnp.pad(o, (0, OMPAD - o.shape[0])) for o in omics])
    w1 = jnp.stack([jnp.pad(w, ((0, OMPAD - w.shape[0]), (0, 0))) for w in w1s])
    h_omic0 = pl.pallas_call(
        _omic_kernel,
        out_shape=jax.ShapeDtypeStruct((8, D), F32),
    )(xom, w1, sig_b1, sig_W2, sig_b2)

    wsi_b2 = wsi_b.reshape(1, D)
    cb = [b.reshape(4, 1, D) for b in (coa_bq, coa_bk, coa_bv, coa_bo, coa_bf)]

    def layer_args(li):
        return (coa_Wq[li], cb[0][li], coa_Wk[li], cb[1][li], coa_Wv[li],
                cb[2][li], coa_Wo[li], cb[3][li], coa_Wf[li], cb[4][li])

    wspec = _full((D, D))
    bspec = _full((1, D))
    lspecs = [wspec, bspec] * 5

    h1, h_omic1 = pl.pallas_call(
        _passA_kernel,
        grid=(NB,),
        in_specs=[
            pl.BlockSpec((BLK, 1024), lambda i: (i, 0)),
            _full((1024, D)), bspec, _full((8, D)),
            *lspecs, *lspecs,
        ],
        out_specs=[
            pl.BlockSpec((BLK, D), lambda i: (i, 0)),
            _full((8, D)),
        ],
        out_shape=[
            jax.ShapeDtypeStruct((N, D), F32),
            jax.ShapeDtypeStruct((8, D), F32),
        ],
        scratch_shapes=[
            pltpu.VMEM((8, D), F32), pltpu.VMEM((8, D), F32),
            pltpu.VMEM((8, D), F32), pltpu.VMEM((64, 128), F32),
            pltpu.VMEM((64, 128), F32), pltpu.VMEM((64, DH), F32),
        ],
    )(x, wsi_W, wsi_b2, h_omic0, *layer_args(0), *layer_args(1))

    clsW_pad = jnp.pad(cls_W, ((0, 0), (0, 128 - cls_W.shape[1])))
    clsb_pad = jnp.pad(cls_b, (0, 128 - cls_b.shape[0])).reshape(1, 128)
    out = pl.pallas_call(
        _passB_kernel,
        grid=(NB,),
        in_specs=[
            pl.BlockSpec((BLK, D), lambda i: (i, 0)),
            _full((8, D)),
            *lspecs, *lspecs,
            _full((2, D, 128)), _full((2, 128)),
            _full((2 * D, 128)), _full((1, 128)),
        ],
        out_specs=_full((8, 128)),
        out_shape=jax.ShapeDtypeStruct((8, 128), F32),
        scratch_shapes=[
            pltpu.VMEM((8, D), F32), pltpu.VMEM((8, D), F32),
            pltpu.VMEM((8, D), F32), pltpu.VMEM((64, 128), F32),
            pltpu.VMEM((64, 128), F32), pltpu.VMEM((64, DH), F32),
            pltpu.VMEM((1, 128), F32), pltpu.VMEM((1, 128), F32),
            pltpu.VMEM((1, D), F32),
        ],
    )(h1, h_omic1, *layer_args(2), *layer_args(3),
      sa_V, sa_w, clsW_pad, clsb_pad)

    return out[0:1, 0:4]


# speedup vs baseline: 2.7412x; 2.7412x over previous
import jax
import jax.numpy as jnp
from jax.experimental import pallas as pl
from jax.experimental.pallas import tpu as pltpu

D = 512
H = 8
N = 16384
BLK = 1024
NB = N // BLK
OMPAD = 640
NEG = -1e30
F32 = jnp.float32
BF = jnp.bfloat16

_SELU_SCALE = 1.0507009873554805
_SELU_ALPHA = 1.6732632423543772


def _selu(x):
    neg = _SELU_ALPHA * (jnp.exp(jnp.minimum(x, 0.0)) - 1.0)
    return _SELU_SCALE * jnp.where(x > 0, x, neg)


def _omic_kernel(xom_ref, w1_ref, b1_ref, w2_ref, b2_ref, out_ref):
    for i in range(6):
        h = xom_ref[i : i + 1, :] @ w1_ref[i]
        h = _selu(h + b1_ref[i : i + 1, :])
        h = _selu(h @ w2_ref[i] + b2_ref[i : i + 1, :])
        out_ref[i : i + 1, :] = h
    out_ref[6:8, :] = jnp.zeros((2, D), F32)


def _dot(a, b):
    return jnp.dot(a, b, preferred_element_type=F32)


def _dot_nt(a, b):
    return jax.lax.dot_general(a, b, (((1,), (1,)), ((), ())),
                               preferred_element_type=F32)


def _path_attn(hb, wqk_ref, sbias_ref, gt_ref, vwf_ref, hbias_ref):
    S = _dot(hb, wqk_ref[...]) + sbias_ref[...]
    M = jnp.max(S, axis=1, keepdims=True)
    E = jnp.exp(S - M)
    Dn = _dot(E, gt_ref[...])
    A = (E / Dn).astype(BF)
    return jnp.maximum(_dot(A, vwf_ref[...]) + hbias_ref[...], 0.0)


def _stream_omic(qw_ref, hb, m_scr, l_scr, u_scr):
    S = _dot_nt(qw_ref[...], hb)
    m_old = m_scr[:, 0:1]
    m_new = jnp.maximum(m_old, jnp.max(S, axis=1, keepdims=True))
    alpha = jnp.exp(m_old - m_new)
    P = jnp.exp(S - m_new)
    l_scr[:, 0:1] = l_scr[:, 0:1] * alpha + jnp.sum(P, axis=1, keepdims=True)
    u_scr[...] = u_scr[...] * alpha + _dot(P.astype(BF), hb)
    m_scr[:, 0:1] = m_new


def _finish_omic(m_scr, l_scr, u_scr, wv_ref, bv_ref, wo_ref, bo_ref,
                 wf_ref, bf_ref):
    an = u_scr[...] / l_scr[:, 0:1]
    z = _dot(an, wv_ref[...])
    o_om = jnp.concatenate(
        [z[h * 8 : (h + 1) * 8, h * 64 : (h + 1) * 64] for h in range(H)],
        axis=1) + bv_ref[...]
    attn = _dot(o_om, wo_ref[...]) + bo_ref[...]
    return jnp.maximum(_dot(attn, wf_ref[...]) + bf_ref[...], 0.0)


def _passA_kernel(x_ref, wsiW_ref, wsib_ref,
                  wqk0_ref, sb0_ref, gt_ref, vwf0_ref, hb0_ref,
                  qw1_ref, wv1_ref, bv1_ref, wo1_ref, bo1_ref, wf1_ref,
                  bf1_ref,
                  h1_ref, hom1_ref,
                  m_scr, l_scr, u_scr):
    i = pl.program_id(0)

    @pl.when(i == 0)
    def _init():
        m_scr[...] = jnp.full(m_scr.shape, NEG, F32)
        l_scr[...] = jnp.zeros(l_scr.shape, F32)
        u_scr[...] = jnp.zeros(u_scr.shape, F32)

    h0 = jnp.maximum(_dot(x_ref[...], wsiW_ref[...]) + wsib_ref[...], 0.0)
    h1 = _path_attn(h0.astype(BF), wqk0_ref, sb0_ref, gt_ref, vwf0_ref,
                    hb0_ref)
    h1b = h1.astype(BF)
    h1_ref[...] = h1b
    _stream_omic(qw1_ref, h1b, m_scr, l_scr, u_scr)

    @pl.when(i == NB - 1)
    def _fin():
        hom1_ref[...] = _finish_omic(m_scr, l_scr, u_scr, wv1_ref, bv1_ref,
                                     wo1_ref, bo1_ref, wf1_ref, bf1_ref)


def _passB_kernel(h1_ref,
                  wqk2_ref, sb2_ref, gt_ref, vwf2_ref, hb2_ref,
                  qw3_ref, wv3_ref, bv3_ref, wo3_ref, bo3_ref, wf3_ref,
                  bf3_ref,
                  sav0_ref, saw0_ref, sav1_ref, saw1_ref,
                  clsW_ref, clsb_ref,
                  out_ref,
                  m_scr, l_scr, u_scr, pm_scr, pl_scr, pacc_scr):
    i = pl.program_id(0)

    @pl.when(i == 0)
    def _init():
        m_scr[...] = jnp.full(m_scr.shape, NEG, F32)
        l_scr[...] = jnp.zeros(l_scr.shape, F32)
        u_scr[...] = jnp.zeros(u_scr.shape, F32)
        pm_scr[...] = jnp.full(pm_scr.shape, NEG, F32)
        pl_scr[...] = jnp.zeros(pl_scr.shape, F32)
        pacc_scr[...] = jnp.zeros(pacc_scr.shape, F32)

    h2 = _path_attn(h1_ref[...], wqk2_ref, sb2_ref, gt_ref, vwf2_ref,
                    hb2_ref)
    h2b = h2.astype(BF)

    t = jnp.tanh(_dot(h2b, sav0_ref[...]))
    sT = _dot_nt(saw0_ref[...], t)
    m_old = pm_scr[0:1, 0:1]
    m_new = jnp.maximum(m_old, jnp.max(sT, axis=1, keepdims=True))
    alpha = jnp.exp(m_old - m_new)
    p = jnp.exp(sT - m_new)
    pl_scr[0:1, 0:1] = pl_scr[0:1, 0:1] * alpha + jnp.sum(
        p, axis=1, keepdims=True)
    pacc_scr[0:1, :] = pacc_scr[0:1, :] * alpha + _dot(p.astype(BF), h2b)
    pm_scr[0:1, 0:1] = m_new

    _stream_omic(qw3_ref, h2b, m_scr, l_scr, u_scr)

    @pl.when(i == NB - 1)
    def _fin():
        hom_f = _finish_omic(m_scr, l_scr, u_scr, wv3_ref, bv3_ref,
                             wo3_ref, bo3_ref, wf3_ref, bf3_ref)
        t2 = jnp.tanh(_dot(hom_f, sav1_ref[...]))
        s2 = _dot_nt(saw1_ref[...], t2)
        mask = jax.lax.broadcasted_iota(jnp.int32, s2.shape, 1) >= 6
        s2 = jnp.where(mask, NEG, s2)
        mx = jnp.max(s2, axis=1, keepdims=True)
        e = jnp.exp(s2 - mx)
        a = e / jnp.sum(e, axis=1, keepdims=True)
        ho = _dot(a, hom_f)
        hp = pacc_scr[0:1, :] / pl_scr[0:1, 0:1]
        hcat = jnp.concatenate([hp, ho], axis=1)
        logits = _dot(hcat, clsW_ref[...]) + clsb_ref[...]
        out_ref[...] = jnp.broadcast_to(logits, out_ref.shape)


def _full(shape):
    return pl.BlockSpec(shape, lambda i: (0,) * len(shape))


def _fold_path_layer(hom6, Wq, bq, Wk, bk, Wv, bv, Wo, bo, Wf, bf):
    k = hom6 @ Wk + bk
    v = hom6 @ Wv + bv
    i8 = jnp.eye(8, dtype=F32)
    kh = jnp.pad(k.reshape(6, 8, 64).transpose(1, 2, 0),
                 ((0, 0), (0, 0), (0, 2)))
    kbd = jnp.einsum('hdj,hH->hdHj', kh, i8).reshape(D, 64)
    wqk = (Wq @ kbd) * 0.125
    lane = jnp.arange(64) % 8
    sbias = ((bq @ kbd) * 0.125 + jnp.where(lane >= 6, NEG, 0.0)
             ).reshape(1, 64)
    vh = jnp.pad(v.reshape(6, 8, 64).transpose(1, 0, 2),
                 ((0, 0), (0, 2), (0, 0)))
    vbd = jnp.einsum('hjd,hH->hjHd', vh, i8).reshape(64, D)
    vwf = vbd @ Wo @ Wf
    hbias = (bo @ Wf + bf).reshape(1, D)
    return wqk.astype(BF), sbias, vwf.astype(BF), hbias


def _fold_omic_queries(hom8, Wq, bq, Wk):
    q = hom8 @ Wq + bq
    qh = q.reshape(8, 8, 64).transpose(1, 0, 2)
    i8 = jnp.eye(8, dtype=F32)
    qbd = jnp.einsum('hqd,hH->hqHd', qh, i8).reshape(64, D)
    return ((qbd @ Wk.T) * 0.125).astype(BF)


def kernel(x_path, x_omic1, x_omic2, x_omic3, x_omic4, x_omic5, x_omic6,
           wsi_W, wsi_b,
           sig_W1_1, sig_W1_2, sig_W1_3, sig_W1_4, sig_W1_5, sig_W1_6,
           sig_b1, sig_W2, sig_b2,
           coa_Wq, coa_bq, coa_Wk, coa_bk, coa_Wv, coa_bv, coa_Wo, coa_bo,
           coa_Wf, coa_bf,
           sa_V, sa_w, cls_W, cls_b):
    xb = x_path[0].astype(BF)

    omics = [x_omic1, x_omic2, x_omic3, x_omic4, x_omic5, x_omic6]
    w1s = [sig_W1_1, sig_W1_2, sig_W1_3, sig_W1_4, sig_W1_5, sig_W1_6]
    xom = jnp.stack([jnp.pad(o, (0, OMPAD - o.shape[0])) for o in omics])
    w1 = jnp.stack([jnp.pad(w, ((0, OMPAD - w.shape[0]), (0, 0)))
                    for w in w1s])
    h_omic0 = pl.pallas_call(
        _omic_kernel,
        out_shape=jax.ShapeDtypeStruct((8, D), F32),
    )(xom, w1, sig_b1, sig_W2, sig_b2)

    gt = (jnp.arange(64)[:, None] // 8 ==
          jnp.arange(64)[None, :] // 8).astype(F32)
    brow = [b.reshape(4, 1, D) for b in (coa_bv, coa_bo, coa_bf)]

    wqk0, sb0, vwf0, hb0 = _fold_path_layer(
        h_omic0[:6], coa_Wq[0], coa_bq[0], coa_Wk[0], coa_bk[0],
        coa_Wv[0], coa_bv[0], coa_Wo[0], coa_bo[0], coa_Wf[0], coa_bf[0])
    qw1 = _fold_omic_queries(h_omic0, coa_Wq[1], coa_bq[1], coa_Wk[1])

    wspec = _full((D, D))
    bspec = _full((1, D))

    h1, h_omic1 = pl.pallas_call(
        _passA_kernel,
        grid=(NB,),
        in_specs=[
            pl.BlockSpec((BLK, 1024), lambda i: (i, 0)),
            _full((1024, D)), bspec,
            _full((D, 64)), _full((1, 64)), _full((64, 64)),
            _full((64, D)), bspec,
            _full((64, D)), wspec, bspec, wspec, bspec, wspec, bspec,
        ],
        out_specs=[
            pl.BlockSpec((BLK, D), lambda i: (i, 0)),
            _full((8, D)),
        ],
        out_shape=[
            jax.ShapeDtypeStruct((N, D), BF),
            jax.ShapeDtypeStruct((8, D), F32),
        ],
        scratch_shapes=[
            pltpu.VMEM((64, 128), F32), pltpu.VMEM((64, 128), F32),
            pltpu.VMEM((64, D), F32),
        ],
    )(xb, wsi_W.astype(BF), wsi_b.reshape(1, D),
      wqk0, sb0, gt, vwf0, hb0,
      qw1, coa_Wv[1], brow[0][1], coa_Wo[1], brow[1][1], coa_Wf[1],
      brow[2][1])

    wqk2, sb2, vwf2, hb2 = _fold_path_layer(
        h_omic1[:6], coa_Wq[2], coa_bq[2], coa_Wk[2], coa_bk[2],
        coa_Wv[2], coa_bv[2], coa_Wo[2], coa_bo[2], coa_Wf[2], coa_bf[2])
    qw3 = _fold_omic_queries(h_omic1, coa_Wq[3], coa_bq[3], coa_Wk[3])

    clsW_pad = jnp.pad(cls_W, ((0, 0), (0, 128 - cls_W.shape[1])))
    clsb_pad = jnp.pad(cls_b, (0, 128 - cls_b.shape[0])).reshape(1, 128)

    out = pl.pallas_call(
        _passB_kernel,
        grid=(NB,),
        in_specs=[
            pl.BlockSpec((BLK, D), lambda i: (i, 0)),
            _full((D, 64)), _full((1, 64)), _full((64, 64)),
            _full((64, D)), bspec,
            _full((64, D)), wspec, bspec, wspec, bspec, wspec, bspec,
            _full((D, 128)), _full((1, 128)), _full((D, 128)),
            _full((1, 128)),
            _full((2 * D, 128)), _full((1, 128)),
        ],
        out_specs=_full((8, 128)),
        out_shape=jax.ShapeDtypeStruct((8, 128), F32),
        scratch_shapes=[
            pltpu.VMEM((64, 128), F32), pltpu.VMEM((64, 128), F32),
            pltpu.VMEM((64, D), F32), pltpu.VMEM((1, 128), F32),
            pltpu.VMEM((1, 128), F32), pltpu.VMEM((1, D), F32),
        ],
    )(h1,
      wqk2, sb2, gt, vwf2, hb2,
      qw3, coa_Wv[3], brow[0][3], coa_Wo[3], brow[1][3], coa_Wf[3],
      brow[2][3],
      sa_V[0].astype(BF), sa_w[0:1], sa_V[1], sa_w[1:2],
      clsW_pad, clsb_pad)

    return out[0:1, 0:4]
